# baseline (speedup 1.0000x reference)
import functools

import jax
import jax.numpy as jnp
from jax import lax
from jax.experimental import pallas as pl
from jax.experimental.pallas import tpu as pltpu
from jax.experimental.pallas import tpu_sc as plsc

VOCAB = 100000
EMBED = 128
CTX = 200
HID = 64

_NC, _NS = 2, 16
_BPW = 8
_NWORK = CTX // _BPW

TILE = 12800
NT = -(-VOCAB // TILE)


def _make_sc_gather():
    mesh = plsc.VectorSubcoreMesh(core_axis_name="c", subcore_axis_name="s")

    @functools.partial(
        pl.kernel,
        mesh=mesh,
        out_type=jax.ShapeDtypeStruct((CTX, EMBED), jnp.float32),
        scratch_types=[
            pltpu.VMEM((_BPW,), jnp.int32),
            pltpu.VMEM((_BPW, EMBED), jnp.float32),
            pltpu.SemaphoreType.DMA,
        ],
    )
    def sc_gather(idx_hbm, table_hbm, out_hbm, idx_v, rows_v, sem):
        wid = lax.axis_index("s") * _NC + lax.axis_index("c")

        @pl.when(wid < _NWORK)
        def _():
            base = wid * _BPW
            pltpu.sync_copy(idx_hbm.at[pl.ds(base, _BPW)], idx_v)
            pltpu.async_copy(table_hbm.at[idx_v], rows_v, sem).wait()
            pltpu.sync_copy(rows_v, out_hbm.at[pl.ds(base, _BPW)])

    return sc_gather


_sc_gather_cache = []


def _sc_gather(idx, table):
    if not _sc_gather_cache:
        _sc_gather_cache.append(_make_sc_gather())
    return _sc_gather_cache[0](idx, table)


def _mlp_body(idx_ref, table_ref, w1_ref, b1_ref, w2t_ref, b2_ref,
              out_ref, logz_ref, e_scr, h_ref, m_ref, l_ref, gsem):
    s = pl.program_id(0)

    @pl.when(s == 0)
    def _():
        def issue(t, c):
            r = idx_ref[t]
            pltpu.make_async_copy(
                table_ref.at[pl.ds(r, 1), :],
                e_scr.at[:, pl.ds(t * EMBED, EMBED)],
                gsem,
            ).start()
            return c

        lax.fori_loop(0, CTX, issue, 0)

        def drain(t, c):
            pltpu.make_async_copy(
                table_ref.at[pl.ds(0, 1), :],
                e_scr.at[:, pl.ds(t * EMBED, EMBED)],
                gsem,
            ).wait()
            return c

        lax.fori_loop(0, CTX, drain, 0)
        h = lax.dot_general(
            e_scr[...], w1_ref[...], (((1,), (1,)), ((), ())),
            preferred_element_type=jnp.float32,
        )
        h_ref[...] = jnp.maximum(h + b1_ref[...], 0.0)
        m_ref[0, 0] = -jnp.inf
        l_ref[0, 0] = 0.0

    o = lax.dot_general(
        h_ref[...], w2t_ref[...], (((1,), (0,)), ((), ())),
        preferred_element_type=jnp.float32,
    ) + b2_ref[...]
    out_ref[...] = o

    col = s * TILE + lax.broadcasted_iota(jnp.int32, o.shape, 1)
    om = jnp.where(col < VOCAB, o, -jnp.inf)
    m_old = m_ref[0, 0]
    m_new = jnp.maximum(m_old, jnp.max(om))
    l_new = l_ref[0, 0] * jnp.exp(m_old - m_new) + jnp.sum(jnp.exp(om - m_new))
    m_ref[0, 0] = m_new
    l_ref[0, 0] = l_new

    @pl.when(s == NT - 1)
    def _():
        logz_ref[0, 0] = m_new + jnp.log(l_new)


def kernel(inputs, table, W1, b1, W2, b2):
    w2t = W2.T

    o_raw, logz = pl.pallas_call(
        _mlp_body,
        grid=(NT,),
        in_specs=[
            pl.BlockSpec(memory_space=pltpu.SMEM),
            pl.BlockSpec(memory_space=pltpu.HBM),
            pl.BlockSpec((HID, CTX * EMBED), lambda s: (0, 0)),
            pl.BlockSpec((1, HID), lambda s: (0, 0)),
            pl.BlockSpec((HID, TILE), lambda s: (0, s)),
            pl.BlockSpec((1, TILE), lambda s: (0, s)),
        ],
        out_specs=[
            pl.BlockSpec((1, TILE), lambda s: (0, s)),
            pl.BlockSpec(memory_space=pltpu.SMEM),
        ],
        out_shape=[
            jax.ShapeDtypeStruct((1, VOCAB), jnp.float32),
            jax.ShapeDtypeStruct((1, 1), jnp.float32),
        ],
        scratch_shapes=[
            pltpu.VMEM((1, CTX * EMBED), jnp.float32),
            pltpu.VMEM((1, HID), jnp.float32),
            pltpu.SMEM((1, 1), jnp.float32),
            pltpu.SMEM((1, 1), jnp.float32),
            pltpu.SemaphoreType.DMA,
        ],
    )(inputs, table, W1, b1.reshape(1, HID), w2t, b2.reshape(1, VOCAB))
    return o_raw - logz


# speedup vs baseline: 1.1057x; 1.1057x over previous
import functools

import jax
import jax.numpy as jnp
from jax import lax
from jax.experimental import pallas as pl
from jax.experimental.pallas import tpu as pltpu
from jax.experimental.pallas import tpu_sc as plsc

VOCAB = 100000
EMBED = 128
CTX = 200
HID = 64

_NC, _NS = 2, 16
_BPW = 8
_NWORK = CTX // _BPW

TILE = 25600
NT = -(-VOCAB // TILE)


def _make_sc_gather():
    mesh = plsc.VectorSubcoreMesh(core_axis_name="c", subcore_axis_name="s")

    @functools.partial(
        pl.kernel,
        mesh=mesh,
        out_type=jax.ShapeDtypeStruct((CTX, EMBED), jnp.float32),
        scratch_types=[
            pltpu.VMEM((_BPW,), jnp.int32),
            pltpu.VMEM((_BPW, EMBED), jnp.float32),
            pltpu.SemaphoreType.DMA,
        ],
    )
    def sc_gather(idx_hbm, table_hbm, out_hbm, idx_v, rows_v, sem):
        wid = lax.axis_index("s") * _NC + lax.axis_index("c")

        @pl.when(wid < _NWORK)
        def _():
            base = wid * _BPW
            pltpu.sync_copy(idx_hbm.at[pl.ds(base, _BPW)], idx_v)
            pltpu.async_copy(table_hbm.at[idx_v], rows_v, sem).wait()
            pltpu.sync_copy(rows_v, out_hbm.at[pl.ds(base, _BPW)])

    return sc_gather


_sc_gather_cache = []


def _sc_gather(idx, table):
    if not _sc_gather_cache:
        _sc_gather_cache.append(_make_sc_gather())
    return _sc_gather_cache[0](idx, table)


def _mlp_body(idx_ref, table_ref, w1_ref, b1_ref, w2t_ref, b2_ref,
              out_ref, logz_ref, e_scr, h_ref, m_ref, l_ref, gsem):
    s = pl.program_id(0)

    @pl.when(s == 0)
    def _():
        def issue(t, c):
            r = idx_ref[t]
            pltpu.make_async_copy(
                table_ref.at[pl.ds(r, 1), :],
                e_scr.at[:, pl.ds(t * EMBED, EMBED)],
                gsem,
            ).start()
            return c

        lax.fori_loop(0, CTX, issue, 0)

        def drain(t, c):
            pltpu.make_async_copy(
                table_ref.at[pl.ds(0, 1), :],
                e_scr.at[:, pl.ds(t * EMBED, EMBED)],
                gsem,
            ).wait()
            return c

        lax.fori_loop(0, CTX, drain, 0)
        h = lax.dot_general(
            e_scr[...], w1_ref[...], (((1,), (1,)), ((), ())),
            preferred_element_type=jnp.float32,
        )
        h_ref[...] = jnp.maximum(h + b1_ref[...], 0.0)
        m_ref[0, 0] = -jnp.inf
        l_ref[0, 0] = 0.0

    o = lax.dot_general(
        h_ref[...], w2t_ref[...], (((1,), (0,)), ((), ())),
        preferred_element_type=jnp.float32,
    ) + b2_ref[...]
    out_ref[...] = o

    col = s * TILE + lax.broadcasted_iota(jnp.int32, o.shape, 1)
    om = jnp.where(col < VOCAB, o, -jnp.inf)
    m_old = m_ref[0, 0]
    m_new = jnp.maximum(m_old, jnp.max(om))
    l_new = l_ref[0, 0] * jnp.exp(m_old - m_new) + jnp.sum(jnp.exp(om - m_new))
    m_ref[0, 0] = m_new
    l_ref[0, 0] = l_new

    @pl.when(s == NT - 1)
    def _():
        logz_ref[0, 0] = m_new + jnp.log(l_new)


def kernel(inputs, table, W1, b1, W2, b2):
    w2t = W2.T

    o_raw, logz = pl.pallas_call(
        _mlp_body,
        grid=(NT,),
        in_specs=[
            pl.BlockSpec(memory_space=pltpu.SMEM),
            pl.BlockSpec(memory_space=pltpu.HBM),
            pl.BlockSpec((HID, CTX * EMBED), lambda s: (0, 0)),
            pl.BlockSpec((1, HID), lambda s: (0, 0)),
            pl.BlockSpec((HID, TILE), lambda s: (0, s)),
            pl.BlockSpec((1, TILE), lambda s: (0, s)),
        ],
        out_specs=[
            pl.BlockSpec((1, TILE), lambda s: (0, s)),
            pl.BlockSpec(memory_space=pltpu.SMEM),
        ],
        out_shape=[
            jax.ShapeDtypeStruct((1, VOCAB), jnp.float32),
            jax.ShapeDtypeStruct((1, 1), jnp.float32),
        ],
        scratch_shapes=[
            pltpu.VMEM((1, CTX * EMBED), jnp.float32),
            pltpu.VMEM((1, HID), jnp.float32),
            pltpu.SMEM((1, 1), jnp.float32),
            pltpu.SMEM((1, 1), jnp.float32),
            pltpu.SemaphoreType.DMA,
        ],
    )(inputs, table, W1, b1.reshape(1, HID), w2t, b2.reshape(1, VOCAB))
    return o_raw - logz


# speedup vs baseline: 1.1172x; 1.0104x over previous
import functools

import jax
import jax.numpy as jnp
from jax import lax
from jax.experimental import pallas as pl
from jax.experimental.pallas import tpu as pltpu
from jax.experimental.pallas import tpu_sc as plsc

VOCAB = 100000
EMBED = 128
CTX = 200
HID = 64

_NC, _NS = 2, 16
_BPW = 8
_NWORK = CTX // _BPW

TILE = 51200
NT = -(-VOCAB // TILE)


def _make_sc_gather():
    mesh = plsc.VectorSubcoreMesh(core_axis_name="c", subcore_axis_name="s")

    @functools.partial(
        pl.kernel,
        mesh=mesh,
        out_type=jax.ShapeDtypeStruct((CTX, EMBED), jnp.float32),
        scratch_types=[
            pltpu.VMEM((_BPW,), jnp.int32),
            pltpu.VMEM((_BPW, EMBED), jnp.float32),
            pltpu.SemaphoreType.DMA,
        ],
    )
    def sc_gather(idx_hbm, table_hbm, out_hbm, idx_v, rows_v, sem):
        wid = lax.axis_index("s") * _NC + lax.axis_index("c")

        @pl.when(wid < _NWORK)
        def _():
            base = wid * _BPW
            pltpu.sync_copy(idx_hbm.at[pl.ds(base, _BPW)], idx_v)
            pltpu.async_copy(table_hbm.at[idx_v], rows_v, sem).wait()
            pltpu.sync_copy(rows_v, out_hbm.at[pl.ds(base, _BPW)])

    return sc_gather


_sc_gather_cache = []


def _sc_gather(idx, table):
    if not _sc_gather_cache:
        _sc_gather_cache.append(_make_sc_gather())
    return _sc_gather_cache[0](idx, table)


def _mlp_body(idx_ref, table_ref, w1_ref, b1_ref, w2t_ref, b2_ref,
              out_ref, logz_ref, e_scr, h_ref, m_ref, l_ref, gsem):
    s = pl.program_id(0)

    @pl.when(s == 0)
    def _():
        def issue(t, c):
            r = idx_ref[t]
            pltpu.make_async_copy(
                table_ref.at[pl.ds(r, 1), :],
                e_scr.at[:, pl.ds(t * EMBED, EMBED)],
                gsem,
            ).start()
            return c

        lax.fori_loop(0, CTX, issue, 0)

        def drain(t, c):
            pltpu.make_async_copy(
                table_ref.at[pl.ds(0, 1), :],
                e_scr.at[:, pl.ds(t * EMBED, EMBED)],
                gsem,
            ).wait()
            return c

        lax.fori_loop(0, CTX, drain, 0)
        h = lax.dot_general(
            e_scr[...], w1_ref[...], (((1,), (1,)), ((), ())),
            preferred_element_type=jnp.float32,
        )
        h_ref[...] = jnp.maximum(h + b1_ref[...], 0.0)
        m_ref[0, 0] = -jnp.inf
        l_ref[0, 0] = 0.0

    o = lax.dot_general(
        h_ref[...], w2t_ref[...], (((1,), (0,)), ((), ())),
        preferred_element_type=jnp.float32,
    ) + b2_ref[...]
    out_ref[...] = o

    col = s * TILE + lax.broadcasted_iota(jnp.int32, o.shape, 1)
    om = jnp.where(col < VOCAB, o, -jnp.inf)
    m_old = m_ref[0, 0]
    m_new = jnp.maximum(m_old, jnp.max(om))
    l_new = l_ref[0, 0] * jnp.exp(m_old - m_new) + jnp.sum(jnp.exp(om - m_new))
    m_ref[0, 0] = m_new
    l_ref[0, 0] = l_new

    @pl.when(s == NT - 1)
    def _():
        logz_ref[0, 0] = m_new + jnp.log(l_new)


def kernel(inputs, table, W1, b1, W2, b2):
    w2t = W2.T

    o_raw, logz = pl.pallas_call(
        _mlp_body,
        grid=(NT,),
        in_specs=[
            pl.BlockSpec(memory_space=pltpu.SMEM),
            pl.BlockSpec(memory_space=pltpu.HBM),
            pl.BlockSpec((HID, CTX * EMBED), lambda s: (0, 0)),
            pl.BlockSpec((1, HID), lambda s: (0, 0)),
            pl.BlockSpec((HID, TILE), lambda s: (0, s)),
            pl.BlockSpec((1, TILE), lambda s: (0, s)),
        ],
        out_specs=[
            pl.BlockSpec((1, TILE), lambda s: (0, s)),
            pl.BlockSpec(memory_space=pltpu.SMEM),
        ],
        out_shape=[
            jax.ShapeDtypeStruct((1, VOCAB), jnp.float32),
            jax.ShapeDtypeStruct((1, 1), jnp.float32),
        ],
        scratch_shapes=[
            pltpu.VMEM((1, CTX * EMBED), jnp.float32),
            pltpu.VMEM((1, HID), jnp.float32),
            pltpu.SMEM((1, 1), jnp.float32),
            pltpu.SMEM((1, 1), jnp.float32),
            pltpu.SemaphoreType.DMA,
        ],
    )(inputs, table, W1, b1.reshape(1, HID), w2t, b2.reshape(1, VOCAB))
    return o_raw - logz
